# Initial kernel scaffold; baseline (speedup 1.0000x reference)
import jax
import jax.numpy as jnp
from jax.experimental import pallas as pl


def kernel(node_features, edge_index, etypes, W0, b0, Wih0, Whh0, bih0, bhh0, W1, b1, Wih1, Whh1, bih1, bhh1, fcW, fcb):
    raise NotImplementedError("write your pallas kernel here")


# speedup vs baseline: 9.3431x; 9.3431x over previous
import functools

import jax
import jax.numpy as jnp
from jax import lax
from jax.experimental import pallas as pl
from jax.experimental.pallas import tpu as pltpu
from jax.experimental.pallas import tpu_sc as plsc

_N = 10000
_H = 128
_E = 320000
_NET = 4
_NSTEPS = 3
_NCLS = 16

_NC = 2
_NS = 16
_NW = _NC * _NS
_K = 128
_NCHUNK = 79
_EPW = _NCHUNK * _K
_EPAD = _NW * _EPW
_NACC = 10112
_ZROW = _NACC // _NS
_OROW = _NACC // _NS

_BN = 1000
_NB = _N // _BN


def _etyp_body(h_ref, wt_ref, b_ref, t_ref):
    t_ref[...] = (
        jnp.dot(h_ref[...], wt_ref[0], preferred_element_type=jnp.float32)
        + b_ref[0]
    )


def _etype_linear(h, wt, b3):
    return pl.pallas_call(
        _etyp_body,
        grid=(_NET, _NB),
        in_specs=[
            pl.BlockSpec((_BN, _H), lambda e, i: (i, 0)),
            pl.BlockSpec((1, _H, _H), lambda e, i: (e, 0, 0)),
            pl.BlockSpec((1, 1, _H), lambda e, i: (e, 0, 0)),
        ],
        out_specs=pl.BlockSpec((_BN, _H), lambda e, i: (e * _NB + i, 0)),
        out_shape=jax.ShapeDtypeStruct((_NET * _N, _H), jnp.float32),
    )(h, wt, b3)


def _gru_body(ap_ref, h_ref, wih_ref, whh_ref, bih_ref, bhh_ref, o_ref):
    a = ap_ref[0] + ap_ref[1]
    h = h_ref[...]
    gi = jnp.dot(a, wih_ref[...], preferred_element_type=jnp.float32) + bih_ref[...]
    gh = jnp.dot(h, whh_ref[...], preferred_element_type=jnp.float32) + bhh_ref[...]
    r = jax.nn.sigmoid(gi[:, :_H] + gh[:, :_H])
    z = jax.nn.sigmoid(gi[:, _H:2 * _H] + gh[:, _H:2 * _H])
    n = jnp.tanh(gi[:, 2 * _H:] + r * gh[:, 2 * _H:])
    o_ref[...] = (1.0 - z) * n + z * h


def _gru(ap, h, wiht, whht, bih2, bhh2):
    return pl.pallas_call(
        _gru_body,
        grid=(_NB,),
        in_specs=[
            pl.BlockSpec((_NC, _BN, _H), lambda i: (0, i, 0)),
            pl.BlockSpec((_BN, _H), lambda i: (i, 0)),
            pl.BlockSpec((_H, 3 * _H), lambda i: (0, 0)),
            pl.BlockSpec((_H, 3 * _H), lambda i: (0, 0)),
            pl.BlockSpec((1, 3 * _H), lambda i: (0, 0)),
            pl.BlockSpec((1, 3 * _H), lambda i: (0, 0)),
        ],
        out_specs=pl.BlockSpec((_BN, _H), lambda i: (i, 0)),
        out_shape=jax.ShapeDtypeStruct((_N, _H), jnp.float32),
    )(ap, h, wiht, whht, bih2, bhh2)


def _pool_body(h_ref, w_ref, b_ref, o_ref):
    pooled = jnp.mean(h_ref[...], axis=0, keepdims=True)
    o_ref[...] = (
        jnp.dot(pooled, w_ref[...], preferred_element_type=jnp.float32)
        + b_ref[...]
    )


def _pool_fc(h, fcwt, fcb2):
    return pl.pallas_call(
        _pool_body,
        out_shape=jax.ShapeDtypeStruct((1, _NCLS), jnp.float32),
    )(h, fcwt, fcb2)


@functools.cache
def _sc_message_pass_fn():
    mesh = plsc.VectorSubcoreMesh(core_axis_name="c", subcore_axis_name="s")

    @functools.partial(
        pl.kernel,
        mesh=mesh,
        out_type=jax.ShapeDtypeStruct((_NC, _NACC, _H), jnp.float32),
        scratch_types=[
            pltpu.VMEM((_K,), jnp.int32),
            pltpu.VMEM((_K,), jnp.int32),
            pltpu.VMEM((_K, _H), jnp.float32),
            pltpu.VMEM_SHARED((_NACC, _H), jnp.float32),
            pltpu.SemaphoreType.DMA,
        ],
    )
    def _sc_message_pass(t_hbm, gidx_hbm, didx_hbm, zeros_hbm, out_hbm,
                         gv, dv, rows, acc, sem):
        c = lax.axis_index("c")
        s = lax.axis_index("s")
        wid = s * _NC + c
        pltpu.sync_copy(zeros_hbm.at[pl.ds(s * _ZROW, _ZROW)],
                        acc.at[pl.ds(s * _ZROW, _ZROW)])
        plsc.subcore_barrier()
        base = wid * _EPW

        def body(j, carry):
            off = base + j * _K
            pltpu.sync_copy(gidx_hbm.at[pl.ds(off, _K)], gv)
            pltpu.sync_copy(didx_hbm.at[pl.ds(off, _K)], dv)
            pltpu.async_copy(t_hbm.at[gv], rows, sem).wait()
            pltpu.sync_copy(rows, acc.at[dv], add=True)
            return carry

        lax.fori_loop(0, _NCHUNK, body, 0)
        plsc.subcore_barrier()
        pltpu.sync_copy(acc.at[pl.ds(s * _OROW, _OROW)],
                        out_hbm.at[c, pl.ds(s * _OROW, _OROW)])

    return _sc_message_pass


def kernel(node_features, edge_index, etypes, W0, b0, Wih0, Whh0, bih0, bhh0,
           W1, b1, Wih1, Whh1, bih1, bhh1, fcW, fcb):
    src = edge_index[0]
    dst = edge_index[1]
    gidx = etypes * _N + src
    pad = _EPAD - _E
    gidx = jnp.concatenate([gidx, jnp.zeros((pad,), jnp.int32)])
    didx = jnp.concatenate([dst, jnp.full((pad,), _N, jnp.int32)])
    zeros = jnp.zeros((_NACC, _H), jnp.float32)

    h = node_features
    for (W, b, Wih, Whh, bih, bhh) in (
        (W0, b0, Wih0, Whh0, bih0, bhh0),
        (W1, b1, Wih1, Whh1, bih1, bhh1),
    ):
        wt = jnp.transpose(W, (0, 2, 1))
        b3 = b[:, None, :]
        wiht = Wih.T
        whht = Whh.T
        bih2 = bih[None, :]
        bhh2 = bhh[None, :]
        for _ in range(_NSTEPS):
            t = _etype_linear(h, wt, b3)
            ap = _sc_message_pass_fn()(t, gidx, didx, zeros)
            h = _gru(ap, h, wiht, whht, bih2, bhh2)

    return _pool_fc(h, fcW.T, fcb[None, :])
